# baseline (speedup 1.0000x reference)
import jax
import jax.numpy as jnp
from jax import lax
from jax.experimental import pallas as pl
from jax.experimental.pallas import tpu as pltpu
from jax.experimental.pallas import tpu_sc as plsc

N_NODES = 10000
N_EDGES = 320000
NC = 2
NS = 16
NW = NC * NS
L = 16

HP = 10240
ZCH = HP // NS
E_SC = N_EDGES // NS
E_W = N_EDGES // NW
E_SC_PAD = E_SC + 96
E_W_PAD = E_W + 112
UZ = 10
UA = 10
UG = 5


def _body(pos_hbm, tail_hbm, out_hbm, pidx_v, tidx_v, hist_v, out_v,
          chunk_v, stage_sh, hist_sh, sem, sem2):
    core = lax.axis_index("c")
    sub = lax.axis_index("s")
    wid = sub * NC + core

    ones = jnp.ones((L,), jnp.float32)
    zeros = jnp.zeros((L,), jnp.float32)

    p_col = pl.multiple_of((sub * E_SC) // 128 * 128, 128)
    p_off = sub * E_SC - p_col
    load_pidx = pltpu.async_copy(
        pos_hbm.at[:, pl.ds(p_col, E_SC_PAD)], pidx_v, sem)

    def zero_step(j, _):
        base = j * (L * UZ)
        for u in range(UZ):
            hist_v[pl.ds(base + u * L, L)] = zeros
        return 0

    with jax.named_scope("ph0_zero_load"):
        lax.fori_loop(0, HP // (L * UZ), zero_step, 0)
        load_pidx.wait()

    load_tidx = pltpu.async_copy(
        tail_hbm.at[pl.ds(wid * E_W, E_W)], tidx_v, sem2)

    def add_step(i, _):
        base = p_off + i * (L * UA)
        for u in range(UA):
            idx = pidx_v[1, pl.ds(base + u * L, L)]
            plsc.addupdate_scatter(hist_v, [idx], ones)
        return 0

    with jax.named_scope("ph1_add"):
        lax.fori_loop(0, E_SC // (L * UA), add_step, 0)

    with jax.named_scope("ph2_merge"):
        pltpu.sync_copy(hist_v, stage_sh.at[sub])
        plsc.subcore_barrier()

        fetch = [
            pltpu.async_copy(stage_sh.at[t, pl.ds(sub * ZCH, ZCH)],
                             chunk_v.at[t], sem)
            for t in range(NS)
        ]
        for d in fetch:
            d.wait()

        def red_step(j, _):
            acc = chunk_v[0, pl.ds(j * L, L)]
            for t in range(1, NS):
                acc = acc + chunk_v[t, pl.ds(j * L, L)]
            hist_v[pl.ds(j * L, L)] = acc
            return 0

        lax.fori_loop(0, ZCH // L, red_step, 0)

        pltpu.sync_copy(hist_v.at[pl.ds(0, ZCH)],
                        hist_sh.at[pl.ds(sub * ZCH, ZCH)])
        plsc.subcore_barrier()

    with jax.named_scope("ph3_gather"):
        pltpu.sync_copy(hist_sh, hist_v)
        load_tidx.wait()

        def gather_step(i, _):
            base = i * (L * UG)
            for u in range(UG):
                idx = tidx_v[pl.ds(base + u * L, L)]
                out_v[pl.ds(base + u * L, L)] = plsc.load_gather(hist_v, [idx])
            return 0

        lax.fori_loop(0, E_W // (L * UG), gather_step, 0)

        pltpu.sync_copy(out_v, out_hbm.at[pl.ds(wid * E_W, E_W)])


_sc_kernel = pl.kernel(
    _body,
    out_type=jax.ShapeDtypeStruct((N_EDGES,), jnp.float32),
    mesh=plsc.VectorSubcoreMesh(core_axis_name="c", subcore_axis_name="s"),
    compiler_params=pltpu.CompilerParams(needs_layout_passes=False),
    scratch_types=[
        pltpu.VMEM((2, E_SC_PAD), jnp.int32),
        pltpu.VMEM((E_W,), jnp.int32),
        pltpu.VMEM((HP,), jnp.float32),
        pltpu.VMEM((E_W,), jnp.float32),
        pltpu.VMEM((NS, ZCH), jnp.float32),
        pltpu.VMEM_SHARED((NS, HP), jnp.float32),
        pltpu.VMEM_SHARED((HP,), jnp.float32),
        pltpu.SemaphoreType.DMA,
        pltpu.SemaphoreType.DMA,
    ],
)


@jax.jit
def kernel(z, edge_index, pos_edge_index):
    del z
    return _sc_kernel(pos_edge_index.astype(jnp.int32),
                      edge_index[1].astype(jnp.int32))


# speedup vs baseline: 1.2708x; 1.2708x over previous
import jax
import jax.numpy as jnp
from jax import lax
from jax.experimental import pallas as pl
from jax.experimental.pallas import tpu as pltpu
from jax.experimental.pallas import tpu_sc as plsc

N_NODES = 10000
N_EDGES = 320000
NC = 2
NS = 16
NW = NC * NS
L = 16

HP = 10240
ZCH = HP // NS
E_SC = N_EDGES // NS
E_W = N_EDGES // NW
E_SC_PAD = E_SC + 96
E_W_PAD = E_W + 112
UZ = 10
UA = 10
UG = 5


def _body(pos_hbm, tail_hbm, out_hbm, pidx_v, tidx_v, hist_v, out_v,
          chunk_v, stage_sh, hist_sh, sem, sem2):
    core = lax.axis_index("c")
    sub = lax.axis_index("s")
    wid = sub * NC + core

    ones = jnp.ones((L,), jnp.float32)
    zeros = jnp.zeros((L,), jnp.float32)

    p_col = pl.multiple_of((sub * E_SC) // 128 * 128, 128)
    p_off = sub * E_SC - p_col
    load_pidx = pltpu.async_copy(
        pos_hbm.at[:, pl.ds(p_col, E_SC_PAD)], pidx_v, sem)

    def zero_step(j, _):
        base = j * (L * UZ)
        for u in range(UZ):
            hist_v[pl.ds(base + u * L, L)] = zeros
        return 0

    with jax.named_scope("ph0_zero_load"):
        lax.fori_loop(0, HP // (L * UZ), zero_step, 0)
        load_pidx.wait()

    load_tidx = pltpu.async_copy(
        tail_hbm.at[pl.ds(N_EDGES + wid * E_W, E_W)], tidx_v, sem2)

    def add_step(i, _):
        base = p_off + i * (L * UA)
        for u in range(UA):
            idx = pidx_v[1, pl.ds(base + u * L, L)]
            plsc.addupdate_scatter(hist_v, [idx], ones)
        return 0

    with jax.named_scope("ph1_add"):
        lax.fori_loop(0, E_SC // (L * UA), add_step, 0)

    with jax.named_scope("ph2_merge"):
        pltpu.sync_copy(hist_v, stage_sh.at[sub])
        plsc.subcore_barrier()

        fetch = [
            pltpu.async_copy(stage_sh.at[t, pl.ds(sub * ZCH, ZCH)],
                             chunk_v.at[t], sem)
            for t in range(NS)
        ]
        for d in fetch:
            d.wait()

        def red_step(j, _):
            acc = chunk_v[0, pl.ds(j * L, L)]
            for t in range(1, NS):
                acc = acc + chunk_v[t, pl.ds(j * L, L)]
            hist_v[pl.ds(j * L, L)] = acc
            return 0

        lax.fori_loop(0, ZCH // L, red_step, 0)

        pltpu.sync_copy(hist_v.at[pl.ds(0, ZCH)],
                        hist_sh.at[pl.ds(sub * ZCH, ZCH)])
        plsc.subcore_barrier()

    with jax.named_scope("ph3_gather"):
        pltpu.sync_copy(hist_sh, hist_v)
        load_tidx.wait()

        def gather_step(i, _):
            base = i * (L * UG)
            for u in range(UG):
                idx = tidx_v[pl.ds(base + u * L, L)]
                out_v[pl.ds(base + u * L, L)] = plsc.load_gather(hist_v, [idx])
            return 0

        lax.fori_loop(0, E_W // (L * UG), gather_step, 0)

        pltpu.sync_copy(out_v, out_hbm.at[pl.ds(wid * E_W, E_W)])


_sc_kernel = pl.kernel(
    _body,
    out_type=jax.ShapeDtypeStruct((N_EDGES,), jnp.float32),
    mesh=plsc.VectorSubcoreMesh(core_axis_name="c", subcore_axis_name="s"),
    compiler_params=pltpu.CompilerParams(needs_layout_passes=False),
    scratch_types=[
        pltpu.VMEM((2, E_SC_PAD), jnp.int32),
        pltpu.VMEM((E_W,), jnp.int32),
        pltpu.VMEM((HP,), jnp.float32),
        pltpu.VMEM((E_W,), jnp.float32),
        pltpu.VMEM((NS, ZCH), jnp.float32),
        pltpu.VMEM_SHARED((NS, HP), jnp.float32),
        pltpu.VMEM_SHARED((HP,), jnp.float32),
        pltpu.SemaphoreType.DMA,
        pltpu.SemaphoreType.DMA,
    ],
)


@jax.jit
def kernel(z, edge_index, pos_edge_index):
    del z
    return _sc_kernel(pos_edge_index.astype(jnp.int32),
                      edge_index.astype(jnp.int32).reshape(-1))
